# baseline (speedup 1.0000x reference)
import jax
import jax.numpy as jnp
from jax.experimental import pallas as pl
from jax.experimental.pallas import tpu as pltpu

_BN = 16384
_C = 128


def _tmk_block(xb_ref, ptsb_ref, c_ref, out_ref, sel_ref):
    DM = sel_ref.shape[0]
    M = c_ref.shape[0]
    D = DM // M

    @pl.when(pl.program_id(0) == 0)
    def _build_sel():
        ri = jax.lax.broadcasted_iota(jnp.int32, (DM, DM), 0)
        ki = jax.lax.broadcasted_iota(jnp.int32, (DM, DM), 1)
        tgt = (ki % _C) * D + (ki // _C)
        sel_ref[...] = jnp.where(ri == tgt, 1.0, 0.0)

    c = c_ref[...]
    xt = jnp.dot(
        xb_ref[...],
        sel_ref[...],
        preferred_element_type=jnp.float32,
        precision=jax.lax.Precision.HIGHEST,
    )
    for j in range(_BN // _C):
        acc = None
        for d in range(D):
            xr = xt[j : j + 1, d * M : (d + 1) * M]
            pb = ptsb_ref[d * M : (d + 1) * M, :]
            t = jnp.abs(pb - xr)
            acc = t if acc is None else acc + t
        kt = jnp.exp(-acc)
        out_ref[j * _C : (j + 1) * _C, :] = jax.lax.dot_general(
            kt, c, (((0,), (0,)), ((), ())), preferred_element_type=jnp.float32
        )


def kernel(input, pts_set, chol_inv):
    N, D = input.shape
    M = pts_set.shape[0]
    xr = input.reshape(N // _C, _C * D)
    pts_b = jnp.broadcast_to(pts_set.T[:, :, None], (D, M, _C)).reshape(D * M, _C)
    return pl.pallas_call(
        _tmk_block,
        grid=(N // _BN,),
        in_specs=[
            pl.BlockSpec((_BN // _C, _C * D), lambda i: (i, 0)),
            pl.BlockSpec((D * M, _C), lambda i: (0, 0)),
            pl.BlockSpec((M, M), lambda i: (0, 0)),
        ],
        out_specs=pl.BlockSpec((_BN, M), lambda i: (i, 0)),
        out_shape=jax.ShapeDtypeStruct((N, M), jnp.float32),
        scratch_shapes=[pltpu.VMEM((D * M, D * M), jnp.float32)],
    )(xr, pts_b, chol_inv)


# speedup vs baseline: 2.3055x; 2.3055x over previous
import jax
import jax.numpy as jnp
from jax.experimental import pallas as pl

_BN = 16384
_C = 128


def _tmk_block(xt_ref, ptsb_ref, c_ref, out_ref):
    D = xt_ref.shape[0]
    M = c_ref.shape[0]
    c = c_ref[...]
    for j in range(_BN // _C):
        acc = None
        for d in range(D):
            xr = xt_ref[d : d + 1, j * _C : (j + 1) * _C]
            pb = ptsb_ref[d * M : (d + 1) * M, :]
            t = jnp.abs(pb - xr)
            acc = t if acc is None else acc + t
        kt = jnp.exp(-acc)
        out_ref[j * _C : (j + 1) * _C, :] = jax.lax.dot_general(
            kt, c, (((0,), (0,)), ((), ())), preferred_element_type=jnp.float32
        )


def kernel(input, pts_set, chol_inv):
    N, D = input.shape
    M = pts_set.shape[0]
    xt = input.T
    pts_b = jnp.broadcast_to(pts_set.T[:, :, None], (D, M, _C)).reshape(D * M, _C)
    return pl.pallas_call(
        _tmk_block,
        grid=(N // _BN,),
        in_specs=[
            pl.BlockSpec((D, _BN), lambda i: (0, i)),
            pl.BlockSpec((D * M, _C), lambda i: (0, 0)),
            pl.BlockSpec((M, M), lambda i: (0, 0)),
        ],
        out_specs=pl.BlockSpec((_BN, M), lambda i: (i, 0)),
        out_shape=jax.ShapeDtypeStruct((N, M), jnp.float32),
    )(xt, pts_b, chol_inv)
